# Initial kernel scaffold; baseline (speedup 1.0000x reference)
import jax
import jax.numpy as jnp
from jax.experimental import pallas as pl


def kernel(x, edge_index, W1, b1, W2, b2, W3, b3, Wm1, bm1, Wm2, bm2):
    raise NotImplementedError("write your pallas kernel here")


# speedup vs baseline: 6.2715x; 6.2715x over previous
import dataclasses
import functools

import jax
import jax.numpy as jnp
from jax import lax
from jax.experimental import pallas as pl
from jax.experimental.pallas import tpu as pltpu
from jax.experimental.pallas import tpu_sc as plsc

N = 10000
D = 256
DH = 128
NSUB = 16
NCORE = 2
NPAD = 10240
ROWS_PER_SUB = NPAD // NSUB
NODES_PER_TILE = NPAD // (NSUB * NCORE)
CHUNK = 128
NCHUNK = 80
EPS = NSUB * NCHUNK * CHUNK
TRASH = NPAD - 1

_mesh = plsc.VectorSubcoreMesh(core_axis_name="c", subcore_axis_name="s")


def _sc_compiler_params():
    cp = pltpu.CompilerParams()
    fields = pltpu.CompilerParams.__dataclass_fields__
    if "needs_layout_passes" in fields:
        cp = dataclasses.replace(cp, needs_layout_passes=False)
    if "use_tc_tiling_on_sc" in fields:
        cp = dataclasses.replace(cp, use_tc_tiling_on_sc=False)
    return cp


def _sc_dinv(dst2d):
    eps = dst2d.shape[1]

    @functools.partial(
        pl.kernel,
        out_type=jax.ShapeDtypeStruct((NPAD,), jnp.float32),
        mesh=_mesh,
        scratch_types=[
            pltpu.VMEM((NPAD,), jnp.float32),
            pltpu.VMEM((eps,), jnp.int32),
            pltpu.VMEM_SHARED((NSUB, NPAD), jnp.float32),
            pltpu.VMEM((NSUB, NODES_PER_TILE), jnp.float32),
            pltpu.VMEM((NODES_PER_TILE,), jnp.float32),
        ],
        compiler_params=_sc_compiler_params(),
    )
    def k(dst_hbm, dinv_hbm, hist_v, idx_v, part_sh, red_v, dv_v):
        c = lax.axis_index("c")
        s = lax.axis_index("s")
        zero16 = jnp.zeros((16,), jnp.float32)
        ones16 = jnp.full((16,), 1.0, jnp.float32)

        @pl.loop(0, NPAD, step=16)
        def _(i):
            hist_v[pl.ds(i, 16)] = zero16

        pltpu.sync_copy(dst_hbm.at[s], idx_v)

        @pl.loop(0, eps, step=16)
        def _(i):
            idx16 = idx_v[pl.ds(i, 16)]
            plsc.addupdate_scatter(hist_v, [idx16], ones16)

        pltpu.sync_copy(hist_v, part_sh.at[s])
        plsc.subcore_barrier()

        base = (c * NSUB + s) * NODES_PER_TILE
        for r in range(NSUB):
            pltpu.sync_copy(part_sh.at[r, pl.ds(base, NODES_PER_TILE)],
                            red_v.at[r])

        @pl.loop(0, NODES_PER_TILE, step=16)
        def _(j):
            acc = jnp.full((16,), 1.0, jnp.float32)
            for r in range(NSUB):
                acc = acc + red_v[r, pl.ds(j, 16)]
            bits = plsc.bitcast(acc, jnp.int32)
            y = plsc.bitcast(jnp.int32(0x5F3759DF) - (bits >> 1), jnp.float32)
            for _ in range(3):
                y = y * (1.5 - 0.5 * acc * y * y)
            dv_v[pl.ds(j, 16)] = y

        pltpu.sync_copy(dv_v, dinv_hbm.at[pl.ds(base, NODES_PER_TILE)])

    return k(dst2d)


def _sc_layer(hlo, hhi, src3d, dst3d, zeros_tab):
    @functools.partial(
        pl.kernel,
        out_type=[jax.ShapeDtypeStruct((NPAD, DH), jnp.float32),
                  jax.ShapeDtypeStruct((NPAD, DH), jnp.float32)],
        mesh=_mesh,
        scratch_types=[
            pltpu.VMEM((NCHUNK, CHUNK), jnp.int32),
            pltpu.VMEM((NCHUNK, CHUNK), jnp.int32),
            pltpu.VMEM((CHUNK, DH), jnp.float32),
            pltpu.VMEM_SHARED((NPAD, DH), jnp.float32),
            pltpu.SemaphoreType.DMA,
        ],
        compiler_params=_sc_compiler_params(),
    )
    def k(hlo_hbm, hhi_hbm, src_hbm, dst_hbm, z_hbm, slo_hbm, shi_hbm,
          src_v, dst_v, rows_v, acc_sh, sem):
        c = lax.axis_index("c")
        s = lax.axis_index("s")
        row0 = s * ROWS_PER_SUB

        pltpu.sync_copy(z_hbm.at[pl.ds(row0, ROWS_PER_SUB)],
                        acc_sh.at[pl.ds(row0, ROWS_PER_SUB)])
        pltpu.sync_copy(src_hbm.at[s], src_v)
        pltpu.sync_copy(dst_hbm.at[s], dst_v)
        plsc.subcore_barrier()

        def run(tab_hbm):
            @pl.loop(0, NCHUNK)
            def _(kk):
                pltpu.async_copy(tab_hbm.at[src_v.at[kk]], rows_v, sem).wait()
                pltpu.sync_copy(rows_v, acc_sh.at[dst_v.at[kk]], add=True)

        @pl.when(c == 0)
        def _():
            run(hlo_hbm)

        @pl.when(c == 1)
        def _():
            run(hhi_hbm)

        plsc.subcore_barrier()

        @pl.when(c == 0)
        def _():
            pltpu.sync_copy(acc_sh.at[pl.ds(row0, ROWS_PER_SUB)],
                            slo_hbm.at[pl.ds(row0, ROWS_PER_SUB)])

        @pl.when(c == 1)
        def _():
            pltpu.sync_copy(acc_sh.at[pl.ds(row0, ROWS_PER_SUB)],
                            shi_hbm.at[pl.ds(row0, ROWS_PER_SUB)])

    return k(hlo, hhi, src3d, dst3d, zeros_tab)


RB = 1000
NRB = N // RB


def _stage_first(x, W1, dv):
    def body(x_ref, w_ref, dv_ref, lo_ref, hi_ref):
        h = jnp.dot(x_ref[...], w_ref[...], preferred_element_type=jnp.float32)
        h = h * dv_ref[...]
        lo_ref[...] = h[:, :DH]
        hi_ref[...] = h[:, DH:]

    return pl.pallas_call(
        body,
        grid=(NRB,),
        in_specs=[
            pl.BlockSpec((RB, D), lambda i: (i, 0)),
            pl.BlockSpec((D, D), lambda i: (0, 0)),
            pl.BlockSpec((RB, 1), lambda i: (i, 0)),
        ],
        out_specs=[
            pl.BlockSpec((RB, DH), lambda i: (i, 0)),
            pl.BlockSpec((RB, DH), lambda i: (i, 0)),
        ],
        out_shape=[jax.ShapeDtypeStruct((N, DH), jnp.float32),
                   jax.ShapeDtypeStruct((N, DH), jnp.float32)],
    )(x, W1, dv)


def _stage_mid(slo, shi, hplo, hphi, dv, b, W):
    def body(slo_ref, shi_ref, plo_ref, phi_ref, dv_ref, b_ref, w_ref,
             lo_ref, hi_ref):
        S = jnp.concatenate([slo_ref[...], shi_ref[...]], axis=1)
        hp = jnp.concatenate([plo_ref[...], phi_ref[...]], axis=1)
        dvb = dv_ref[...]
        conv = (S + hp) * dvb + b_ref[...]
        h = jnp.maximum(conv, 0.0)
        out = jnp.dot(h, w_ref[...], preferred_element_type=jnp.float32) * dvb
        lo_ref[...] = out[:, :DH]
        hi_ref[...] = out[:, DH:]

    return pl.pallas_call(
        body,
        grid=(NRB,),
        in_specs=[
            pl.BlockSpec((RB, DH), lambda i: (i, 0)),
            pl.BlockSpec((RB, DH), lambda i: (i, 0)),
            pl.BlockSpec((RB, DH), lambda i: (i, 0)),
            pl.BlockSpec((RB, DH), lambda i: (i, 0)),
            pl.BlockSpec((RB, 1), lambda i: (i, 0)),
            pl.BlockSpec((1, D), lambda i: (0, 0)),
            pl.BlockSpec((D, D), lambda i: (0, 0)),
        ],
        out_specs=[
            pl.BlockSpec((RB, DH), lambda i: (i, 0)),
            pl.BlockSpec((RB, DH), lambda i: (i, 0)),
        ],
        out_shape=[jax.ShapeDtypeStruct((N, DH), jnp.float32),
                   jax.ShapeDtypeStruct((N, DH), jnp.float32)],
    )(slo, shi, hplo, hphi, dv, b, W)


def _stage_final(slo, shi, hplo, hphi, dv, b3, Wm1, bm1, Wm2, bm2):
    def body(slo_ref, shi_ref, plo_ref, phi_ref, dv_ref, b3_ref,
             wm1_ref, bm1_ref, wm2_ref, bm2_ref, o_ref):
        S = jnp.concatenate([slo_ref[...], shi_ref[...]], axis=1)
        hp = jnp.concatenate([plo_ref[...], phi_ref[...]], axis=1)
        conv = (S + hp) * dv_ref[...] + b3_ref[...]
        t = jnp.dot(conv, wm1_ref[...], preferred_element_type=jnp.float32)
        t = jnp.maximum(t + bm1_ref[...], 0.0)
        o = jnp.dot(t, wm2_ref[...], preferred_element_type=jnp.float32)
        o_ref[...] = o + bm2_ref[...]

    return pl.pallas_call(
        body,
        grid=(NRB,),
        in_specs=[
            pl.BlockSpec((RB, DH), lambda i: (i, 0)),
            pl.BlockSpec((RB, DH), lambda i: (i, 0)),
            pl.BlockSpec((RB, DH), lambda i: (i, 0)),
            pl.BlockSpec((RB, DH), lambda i: (i, 0)),
            pl.BlockSpec((RB, 1), lambda i: (i, 0)),
            pl.BlockSpec((1, D), lambda i: (0, 0)),
            pl.BlockSpec((D, DH), lambda i: (0, 0)),
            pl.BlockSpec((1, DH), lambda i: (0, 0)),
            pl.BlockSpec((DH, D), lambda i: (0, 0)),
            pl.BlockSpec((1, D), lambda i: (0, 0)),
        ],
        out_specs=pl.BlockSpec((RB, D), lambda i: (i, 0)),
        out_shape=jax.ShapeDtypeStruct((N, D), jnp.float32),
    )(slo, shi, hplo, hphi, dv, b3, Wm1, bm1, Wm2, bm2)


def kernel(x, edge_index, W1, b1, W2, b2, W3, b3, Wm1, bm1, Wm2, bm2):
    src = edge_index[0].astype(jnp.int32)
    dst = edge_index[1].astype(jnp.int32)
    e = src.shape[0]
    pad = EPS - e
    src3d = jnp.concatenate(
        [src, jnp.zeros((pad,), jnp.int32)]).reshape(NSUB, NCHUNK, CHUNK)
    dst3d = jnp.concatenate(
        [dst, jnp.full((pad,), TRASH, jnp.int32)]).reshape(NSUB, NCHUNK, CHUNK)
    dst2d = dst.reshape(NSUB, e // NSUB)

    dinv = _sc_dinv(dst2d)
    dv = dinv[:N].reshape(N, 1)
    zeros_tab = jnp.zeros((NPAD, DH), jnp.float32)
    b1r = b1.reshape(1, D)
    b2r = b2.reshape(1, D)
    b3r = b3.reshape(1, D)
    bm1r = bm1.reshape(1, DH)
    bm2r = bm2.reshape(1, D)

    h1lo, h1hi = _stage_first(x, W1, dv)
    s1lo, s1hi = _sc_layer(h1lo, h1hi, src3d, dst3d, zeros_tab)
    h2lo, h2hi = _stage_mid(s1lo, s1hi, h1lo, h1hi, dv, b1r, W2)
    s2lo, s2hi = _sc_layer(h2lo, h2hi, src3d, dst3d, zeros_tab)
    h3lo, h3hi = _stage_mid(s2lo, s2hi, h2lo, h2hi, dv, b2r, W3)
    s3lo, s3hi = _sc_layer(h3lo, h3hi, src3d, dst3d, zeros_tab)
    return _stage_final(s3lo, s3hi, h3lo, h3hi, dv, b3r, Wm1, bm1r, Wm2, bm2r)


# speedup vs baseline: 6.9203x; 1.1035x over previous
import dataclasses
import functools

import jax
import jax.numpy as jnp
from jax import lax
from jax.experimental import pallas as pl
from jax.experimental.pallas import tpu as pltpu
from jax.experimental.pallas import tpu_sc as plsc

N = 10000
D = 256
DH = 128
NSUB = 16
NCORE = 2
NPAD = 10240
ROWS_PER_SUB = NPAD // NSUB
NODES_PER_TILE = NPAD // (NSUB * NCORE)
CHUNK = 128
NCHUNK = 80
SWEEPS = 2
CPS = NCHUNK // SWEEPS
EPS = NSUB * NCHUNK * CHUNK
TRASH = NPAD - 1

_mesh = plsc.VectorSubcoreMesh(core_axis_name="c", subcore_axis_name="s")


def _sc_compiler_params():
    cp = pltpu.CompilerParams()
    fields = pltpu.CompilerParams.__dataclass_fields__
    if "needs_layout_passes" in fields:
        cp = dataclasses.replace(cp, needs_layout_passes=False)
    if "use_tc_tiling_on_sc" in fields:
        cp = dataclasses.replace(cp, use_tc_tiling_on_sc=False)
    return cp


def _sc_dinv(dst2d):
    eps = dst2d.shape[1]

    @functools.partial(
        pl.kernel,
        out_type=jax.ShapeDtypeStruct((NPAD,), jnp.float32),
        mesh=_mesh,
        scratch_types=[
            pltpu.VMEM((NPAD,), jnp.float32),
            pltpu.VMEM((eps,), jnp.int32),
            pltpu.VMEM_SHARED((NSUB, NPAD), jnp.float32),
            pltpu.VMEM((NSUB, NODES_PER_TILE), jnp.float32),
            pltpu.VMEM((NODES_PER_TILE,), jnp.float32),
        ],
        compiler_params=_sc_compiler_params(),
    )
    def k(dst_hbm, dinv_hbm, hist_v, idx_v, part_sh, red_v, dv_v):
        c = lax.axis_index("c")
        s = lax.axis_index("s")
        zero16 = jnp.zeros((16,), jnp.float32)
        ones16 = jnp.full((16,), 1.0, jnp.float32)

        @pl.loop(0, NPAD, step=16)
        def _(i):
            hist_v[pl.ds(i, 16)] = zero16

        pltpu.sync_copy(dst_hbm.at[s], idx_v)

        @pl.loop(0, eps, step=16)
        def _(i):
            idx16 = idx_v[pl.ds(i, 16)]
            plsc.addupdate_scatter(hist_v, [idx16], ones16)

        pltpu.sync_copy(hist_v, part_sh.at[s])
        plsc.subcore_barrier()

        base = (c * NSUB + s) * NODES_PER_TILE
        for r in range(NSUB):
            pltpu.sync_copy(part_sh.at[r, pl.ds(base, NODES_PER_TILE)],
                            red_v.at[r])

        @pl.loop(0, NODES_PER_TILE, step=16)
        def _(j):
            acc = jnp.full((16,), 1.0, jnp.float32)
            for r in range(NSUB):
                acc = acc + red_v[r, pl.ds(j, 16)]
            bits = plsc.bitcast(acc, jnp.int32)
            y = plsc.bitcast(jnp.int32(0x5F3759DF) - (bits >> 1), jnp.float32)
            for _ in range(3):
                y = y * (1.5 - 0.5 * acc * y * y)
            dv_v[pl.ds(j, 16)] = y

        pltpu.sync_copy(dv_v, dinv_hbm.at[pl.ds(base, NODES_PER_TILE)])

    return k(dst2d)


def _sc_layer(hlo, hhi, src3d, dst3d, zeros_tab):
    @functools.partial(
        pl.kernel,
        out_type=[jax.ShapeDtypeStruct((NPAD, DH), jnp.float32),
                  jax.ShapeDtypeStruct((NPAD, DH), jnp.float32)],
        mesh=_mesh,
        scratch_types=[
            pltpu.VMEM((CPS, CHUNK), jnp.int32),
            pltpu.VMEM((CPS, CHUNK), jnp.int32),
            pltpu.VMEM((CHUNK, DH), jnp.float32),
            pltpu.VMEM((CHUNK, DH), jnp.float32),
            pltpu.VMEM_SHARED((NPAD, DH), jnp.float32),
            pltpu.SemaphoreType.DMA,
            pltpu.SemaphoreType.DMA,
            pltpu.SemaphoreType.DMA,
            pltpu.SemaphoreType.DMA,
        ],
        compiler_params=_sc_compiler_params(),
    )
    def k(hlo_hbm, hhi_hbm, src_hbm, dst_hbm, z_hbm, slo_hbm, shi_hbm,
          src_v, dst_v, rows0_v, rows1_v, acc_sh, gsem0, gsem1, ssem0, ssem1):
        c = lax.axis_index("c")
        s = lax.axis_index("s")
        row0 = s * ROWS_PER_SUB

        pltpu.sync_copy(z_hbm.at[pl.ds(row0, ROWS_PER_SUB)],
                        acc_sh.at[pl.ds(row0, ROWS_PER_SUB)])
        plsc.subcore_barrier()

        def run(tab_hbm):
            rows = (rows0_v, rows1_v)
            gsem = (gsem0, gsem1)
            ssem = (ssem0, ssem1)

            def gstart(kk, b):
                pltpu.async_copy(
                    tab_hbm.at[src_v.at[kk]], rows[b], gsem[b])

            def gwait(kk, b):
                pltpu.make_async_copy(
                    tab_hbm.at[src_v.at[kk]], rows[b], gsem[b]).wait()

            def sstart(kk, b):
                pltpu.async_copy(
                    rows[b], acc_sh.at[dst_v.at[kk]], ssem[b], add=True)

            def swait(kk, b):
                pltpu.make_async_copy(
                    rows[b], acc_sh.at[dst_v.at[kk]], ssem[b]).wait()

            for sw in range(SWEEPS):
                pltpu.sync_copy(src_hbm.at[s, pl.ds(sw * CPS, CPS)], src_v)
                pltpu.sync_copy(dst_hbm.at[s, pl.ds(sw * CPS, CPS)], dst_v)
                gstart(0, 0)
                gstart(1, 1)

                @pl.loop(0, CPS, step=2)
                def _(kk):
                    gwait(kk, 0)
                    sstart(kk, 0)
                    gwait(kk + 1, 1)
                    sstart(kk + 1, 1)

                    @pl.when(kk + 2 < CPS)
                    def _():
                        swait(kk, 0)
                        gstart(kk + 2, 0)
                        swait(kk + 1, 1)
                        gstart(kk + 3, 1)

                swait(CPS - 2, 0)
                swait(CPS - 1, 1)

        @pl.when(c == 0)
        def _():
            run(hlo_hbm)

        @pl.when(c == 1)
        def _():
            run(hhi_hbm)

        plsc.subcore_barrier()

        @pl.when(c == 0)
        def _():
            pltpu.sync_copy(acc_sh.at[pl.ds(row0, ROWS_PER_SUB)],
                            slo_hbm.at[pl.ds(row0, ROWS_PER_SUB)])

        @pl.when(c == 1)
        def _():
            pltpu.sync_copy(acc_sh.at[pl.ds(row0, ROWS_PER_SUB)],
                            shi_hbm.at[pl.ds(row0, ROWS_PER_SUB)])

    return k(hlo, hhi, src3d, dst3d, zeros_tab)


RB = 1000
NRB = N // RB


def _stage_first(x, W1, dv):
    def body(x_ref, w_ref, dv_ref, lo_ref, hi_ref):
        h = jnp.dot(x_ref[...], w_ref[...], preferred_element_type=jnp.float32)
        h = h * dv_ref[...]
        lo_ref[...] = h[:, :DH]
        hi_ref[...] = h[:, DH:]

    return pl.pallas_call(
        body,
        grid=(NRB,),
        in_specs=[
            pl.BlockSpec((RB, D), lambda i: (i, 0)),
            pl.BlockSpec((D, D), lambda i: (0, 0)),
            pl.BlockSpec((RB, 1), lambda i: (i, 0)),
        ],
        out_specs=[
            pl.BlockSpec((RB, DH), lambda i: (i, 0)),
            pl.BlockSpec((RB, DH), lambda i: (i, 0)),
        ],
        out_shape=[jax.ShapeDtypeStruct((N, DH), jnp.float32),
                   jax.ShapeDtypeStruct((N, DH), jnp.float32)],
    )(x, W1, dv)


def _stage_mid(slo, shi, hplo, hphi, dv, b, W):
    def body(slo_ref, shi_ref, plo_ref, phi_ref, dv_ref, b_ref, w_ref,
             lo_ref, hi_ref):
        S = jnp.concatenate([slo_ref[...], shi_ref[...]], axis=1)
        hp = jnp.concatenate([plo_ref[...], phi_ref[...]], axis=1)
        dvb = dv_ref[...]
        conv = (S + hp) * dvb + b_ref[...]
        h = jnp.maximum(conv, 0.0)
        out = jnp.dot(h, w_ref[...], preferred_element_type=jnp.float32) * dvb
        lo_ref[...] = out[:, :DH]
        hi_ref[...] = out[:, DH:]

    return pl.pallas_call(
        body,
        grid=(NRB,),
        in_specs=[
            pl.BlockSpec((RB, DH), lambda i: (i, 0)),
            pl.BlockSpec((RB, DH), lambda i: (i, 0)),
            pl.BlockSpec((RB, DH), lambda i: (i, 0)),
            pl.BlockSpec((RB, DH), lambda i: (i, 0)),
            pl.BlockSpec((RB, 1), lambda i: (i, 0)),
            pl.BlockSpec((1, D), lambda i: (0, 0)),
            pl.BlockSpec((D, D), lambda i: (0, 0)),
        ],
        out_specs=[
            pl.BlockSpec((RB, DH), lambda i: (i, 0)),
            pl.BlockSpec((RB, DH), lambda i: (i, 0)),
        ],
        out_shape=[jax.ShapeDtypeStruct((N, DH), jnp.float32),
                   jax.ShapeDtypeStruct((N, DH), jnp.float32)],
    )(slo, shi, hplo, hphi, dv, b, W)


def _stage_final(slo, shi, hplo, hphi, dv, b3, Wm1, bm1, Wm2, bm2):
    def body(slo_ref, shi_ref, plo_ref, phi_ref, dv_ref, b3_ref,
             wm1_ref, bm1_ref, wm2_ref, bm2_ref, o_ref):
        S = jnp.concatenate([slo_ref[...], shi_ref[...]], axis=1)
        hp = jnp.concatenate([plo_ref[...], phi_ref[...]], axis=1)
        conv = (S + hp) * dv_ref[...] + b3_ref[...]
        t = jnp.dot(conv, wm1_ref[...], preferred_element_type=jnp.float32)
        t = jnp.maximum(t + bm1_ref[...], 0.0)
        o = jnp.dot(t, wm2_ref[...], preferred_element_type=jnp.float32)
        o_ref[...] = o + bm2_ref[...]

    return pl.pallas_call(
        body,
        grid=(NRB,),
        in_specs=[
            pl.BlockSpec((RB, DH), lambda i: (i, 0)),
            pl.BlockSpec((RB, DH), lambda i: (i, 0)),
            pl.BlockSpec((RB, DH), lambda i: (i, 0)),
            pl.BlockSpec((RB, DH), lambda i: (i, 0)),
            pl.BlockSpec((RB, 1), lambda i: (i, 0)),
            pl.BlockSpec((1, D), lambda i: (0, 0)),
            pl.BlockSpec((D, DH), lambda i: (0, 0)),
            pl.BlockSpec((1, DH), lambda i: (0, 0)),
            pl.BlockSpec((DH, D), lambda i: (0, 0)),
            pl.BlockSpec((1, D), lambda i: (0, 0)),
        ],
        out_specs=pl.BlockSpec((RB, D), lambda i: (i, 0)),
        out_shape=jax.ShapeDtypeStruct((N, D), jnp.float32),
    )(slo, shi, hplo, hphi, dv, b3, Wm1, bm1, Wm2, bm2)


def kernel(x, edge_index, W1, b1, W2, b2, W3, b3, Wm1, bm1, Wm2, bm2):
    src = edge_index[0].astype(jnp.int32)
    dst = edge_index[1].astype(jnp.int32)
    e = src.shape[0]
    pad = EPS - e
    src3d = jnp.concatenate(
        [src, jnp.zeros((pad,), jnp.int32)]).reshape(NSUB, NCHUNK, CHUNK)
    dst3d = jnp.concatenate(
        [dst, jnp.full((pad,), TRASH, jnp.int32)]).reshape(NSUB, NCHUNK, CHUNK)
    dst2d = dst.reshape(NSUB, e // NSUB)

    dinv = _sc_dinv(dst2d)
    dv = dinv[:N].reshape(N, 1)
    zeros_tab = jnp.zeros((NPAD, DH), jnp.float32)
    b1r = b1.reshape(1, D)
    b2r = b2.reshape(1, D)
    b3r = b3.reshape(1, D)
    bm1r = bm1.reshape(1, DH)
    bm2r = bm2.reshape(1, D)

    h1lo, h1hi = _stage_first(x, W1, dv)
    s1lo, s1hi = _sc_layer(h1lo, h1hi, src3d, dst3d, zeros_tab)
    h2lo, h2hi = _stage_mid(s1lo, s1hi, h1lo, h1hi, dv, b1r, W2)
    s2lo, s2hi = _sc_layer(h2lo, h2hi, src3d, dst3d, zeros_tab)
    h3lo, h3hi = _stage_mid(s2lo, s2hi, h2lo, h2hi, dv, b2r, W3)
    s3lo, s3hi = _sc_layer(h3lo, h3hi, src3d, dst3d, zeros_tab)
    return _stage_final(s3lo, s3hi, h3lo, h3hi, dv, b3r, Wm1, bm1r, Wm2, bm2r)


# speedup vs baseline: 7.4030x; 1.0697x over previous
import dataclasses
import functools

import jax
import jax.numpy as jnp
from jax import lax
from jax.experimental import pallas as pl
from jax.experimental.pallas import tpu as pltpu
from jax.experimental.pallas import tpu_sc as plsc

N = 10000
D = 256
DH = 128
NSUB = 16
NCORE = 2
NPAD = 10240
ROWS_PER_SUB = NPAD // NSUB
NODES_PER_TILE = NPAD // (NSUB * NCORE)
CHUNK = 64
NCHUNK = 160
SWEEPS = 4
CPS = NCHUNK // SWEEPS
NBUF = 4
EPS = NSUB * NCHUNK * CHUNK
TRASH = NPAD - 1

_mesh = plsc.VectorSubcoreMesh(core_axis_name="c", subcore_axis_name="s")


def _sc_compiler_params():
    cp = pltpu.CompilerParams()
    fields = pltpu.CompilerParams.__dataclass_fields__
    if "needs_layout_passes" in fields:
        cp = dataclasses.replace(cp, needs_layout_passes=False)
    if "use_tc_tiling_on_sc" in fields:
        cp = dataclasses.replace(cp, use_tc_tiling_on_sc=False)
    return cp


def _sc_dinv(dst2d):
    eps = dst2d.shape[1]

    @functools.partial(
        pl.kernel,
        out_type=jax.ShapeDtypeStruct((NPAD,), jnp.float32),
        mesh=_mesh,
        scratch_types=[
            pltpu.VMEM((NPAD,), jnp.float32),
            pltpu.VMEM((eps,), jnp.int32),
            pltpu.VMEM_SHARED((NSUB, NPAD), jnp.float32),
            pltpu.VMEM((NSUB, NODES_PER_TILE), jnp.float32),
            pltpu.VMEM((NODES_PER_TILE,), jnp.float32),
        ],
        compiler_params=_sc_compiler_params(),
    )
    def k(dst_hbm, dinv_hbm, hist_v, idx_v, part_sh, red_v, dv_v):
        c = lax.axis_index("c")
        s = lax.axis_index("s")
        zero16 = jnp.zeros((16,), jnp.float32)
        ones16 = jnp.full((16,), 1.0, jnp.float32)

        @pl.loop(0, NPAD, step=16)
        def _(i):
            hist_v[pl.ds(i, 16)] = zero16

        pltpu.sync_copy(dst_hbm.at[s], idx_v)

        @pl.loop(0, eps, step=16)
        def _(i):
            idx16 = idx_v[pl.ds(i, 16)]
            plsc.addupdate_scatter(hist_v, [idx16], ones16)

        pltpu.sync_copy(hist_v, part_sh.at[s])
        plsc.subcore_barrier()

        base = (c * NSUB + s) * NODES_PER_TILE
        for r in range(NSUB):
            pltpu.sync_copy(part_sh.at[r, pl.ds(base, NODES_PER_TILE)],
                            red_v.at[r])

        @pl.loop(0, NODES_PER_TILE, step=16)
        def _(j):
            acc = jnp.full((16,), 1.0, jnp.float32)
            for r in range(NSUB):
                acc = acc + red_v[r, pl.ds(j, 16)]
            bits = plsc.bitcast(acc, jnp.int32)
            y = plsc.bitcast(jnp.int32(0x5F3759DF) - (bits >> 1), jnp.float32)
            for _ in range(3):
                y = y * (1.5 - 0.5 * acc * y * y)
            dv_v[pl.ds(j, 16)] = y

        pltpu.sync_copy(dv_v, dinv_hbm.at[pl.ds(base, NODES_PER_TILE)])

    return k(dst2d)


def _sc_layer(hlo, hhi, src3d, dst3d, zeros_tab):
    @functools.partial(
        pl.kernel,
        out_type=[jax.ShapeDtypeStruct((NPAD, DH), jnp.float32),
                  jax.ShapeDtypeStruct((NPAD, DH), jnp.float32)],
        mesh=_mesh,
        scratch_types=(
            [pltpu.VMEM((CPS, CHUNK), jnp.int32),
             pltpu.VMEM((CPS, CHUNK), jnp.int32)]
            + [pltpu.VMEM((CHUNK, DH), jnp.float32) for _ in range(NBUF)]
            + [pltpu.VMEM_SHARED((NPAD, DH), jnp.float32)]
            + [pltpu.SemaphoreType.DMA for _ in range(2 * NBUF)]
        ),
        compiler_params=_sc_compiler_params(),
    )
    def k(hlo_hbm, hhi_hbm, src_hbm, dst_hbm, z_hbm, slo_hbm, shi_hbm,
          src_v, dst_v, *rest):
        rows = rest[:NBUF]
        acc_sh = rest[NBUF]
        gsem = rest[NBUF + 1:2 * NBUF + 1]
        ssem = rest[2 * NBUF + 1:]
        c = lax.axis_index("c")
        s = lax.axis_index("s")
        row0 = s * ROWS_PER_SUB

        pltpu.sync_copy(z_hbm.at[pl.ds(row0, ROWS_PER_SUB)],
                        acc_sh.at[pl.ds(row0, ROWS_PER_SUB)])
        plsc.subcore_barrier()

        def run(tab_hbm):
            def gstart(kk, b):
                pltpu.async_copy(
                    tab_hbm.at[src_v.at[kk]], rows[b], gsem[b])

            def gwait(kk, b):
                pltpu.make_async_copy(
                    tab_hbm.at[src_v.at[kk]], rows[b], gsem[b]).wait()

            def sstart(kk, b):
                pltpu.async_copy(
                    rows[b], acc_sh.at[dst_v.at[kk]], ssem[b], add=True)

            def swait(kk, b):
                pltpu.make_async_copy(
                    rows[b], acc_sh.at[dst_v.at[kk]], ssem[b]).wait()

            for sw in range(SWEEPS):
                pltpu.sync_copy(src_hbm.at[s, pl.ds(sw * CPS, CPS)], src_v)
                pltpu.sync_copy(dst_hbm.at[s, pl.ds(sw * CPS, CPS)], dst_v)
                for r in range(NBUF - 1):
                    gstart(r, r)

                @pl.loop(0, CPS, step=NBUF)
                def _(kk):
                    for r in range(NBUF):
                        j = kk + r
                        bn = (r + NBUF - 1) % NBUF
                        gwait(j, r)
                        sstart(j, r)

                        @pl.when(j + NBUF - 1 < CPS)
                        def _(j=j, bn=bn):
                            @pl.when(j >= 1)
                            def _():
                                swait(j - 1, bn)

                            gstart(j + NBUF - 1, bn)

                for r in range(NBUF):
                    jj = CPS - NBUF + r
                    swait(jj, jj % NBUF)

        @pl.when(c == 0)
        def _():
            run(hlo_hbm)

        @pl.when(c == 1)
        def _():
            run(hhi_hbm)

        plsc.subcore_barrier()

        @pl.when(c == 0)
        def _():
            pltpu.sync_copy(acc_sh.at[pl.ds(row0, ROWS_PER_SUB)],
                            slo_hbm.at[pl.ds(row0, ROWS_PER_SUB)])

        @pl.when(c == 1)
        def _():
            pltpu.sync_copy(acc_sh.at[pl.ds(row0, ROWS_PER_SUB)],
                            shi_hbm.at[pl.ds(row0, ROWS_PER_SUB)])

    return k(hlo, hhi, src3d, dst3d, zeros_tab)


RB = 1000
NRB = N // RB


def _stage_first(x, W1, dv):
    def body(x_ref, w_ref, dv_ref, lo_ref, hi_ref):
        h = jnp.dot(x_ref[...], w_ref[...], preferred_element_type=jnp.float32)
        h = h * dv_ref[...]
        lo_ref[...] = h[:, :DH]
        hi_ref[...] = h[:, DH:]

    return pl.pallas_call(
        body,
        grid=(NRB,),
        in_specs=[
            pl.BlockSpec((RB, D), lambda i: (i, 0)),
            pl.BlockSpec((D, D), lambda i: (0, 0)),
            pl.BlockSpec((RB, 1), lambda i: (i, 0)),
        ],
        out_specs=[
            pl.BlockSpec((RB, DH), lambda i: (i, 0)),
            pl.BlockSpec((RB, DH), lambda i: (i, 0)),
        ],
        out_shape=[jax.ShapeDtypeStruct((N, DH), jnp.float32),
                   jax.ShapeDtypeStruct((N, DH), jnp.float32)],
    )(x, W1, dv)


def _stage_mid(slo, shi, hplo, hphi, dv, b, W):
    def body(slo_ref, shi_ref, plo_ref, phi_ref, dv_ref, b_ref, w_ref,
             lo_ref, hi_ref):
        S = jnp.concatenate([slo_ref[...], shi_ref[...]], axis=1)
        hp = jnp.concatenate([plo_ref[...], phi_ref[...]], axis=1)
        dvb = dv_ref[...]
        conv = (S + hp) * dvb + b_ref[...]
        h = jnp.maximum(conv, 0.0)
        out = jnp.dot(h, w_ref[...], preferred_element_type=jnp.float32) * dvb
        lo_ref[...] = out[:, :DH]
        hi_ref[...] = out[:, DH:]

    return pl.pallas_call(
        body,
        grid=(NRB,),
        in_specs=[
            pl.BlockSpec((RB, DH), lambda i: (i, 0)),
            pl.BlockSpec((RB, DH), lambda i: (i, 0)),
            pl.BlockSpec((RB, DH), lambda i: (i, 0)),
            pl.BlockSpec((RB, DH), lambda i: (i, 0)),
            pl.BlockSpec((RB, 1), lambda i: (i, 0)),
            pl.BlockSpec((1, D), lambda i: (0, 0)),
            pl.BlockSpec((D, D), lambda i: (0, 0)),
        ],
        out_specs=[
            pl.BlockSpec((RB, DH), lambda i: (i, 0)),
            pl.BlockSpec((RB, DH), lambda i: (i, 0)),
        ],
        out_shape=[jax.ShapeDtypeStruct((N, DH), jnp.float32),
                   jax.ShapeDtypeStruct((N, DH), jnp.float32)],
    )(slo, shi, hplo, hphi, dv, b, W)


def _stage_final(slo, shi, hplo, hphi, dv, b3, Wm1, bm1, Wm2, bm2):
    def body(slo_ref, shi_ref, plo_ref, phi_ref, dv_ref, b3_ref,
             wm1_ref, bm1_ref, wm2_ref, bm2_ref, o_ref):
        S = jnp.concatenate([slo_ref[...], shi_ref[...]], axis=1)
        hp = jnp.concatenate([plo_ref[...], phi_ref[...]], axis=1)
        conv = (S + hp) * dv_ref[...] + b3_ref[...]
        t = jnp.dot(conv, wm1_ref[...], preferred_element_type=jnp.float32)
        t = jnp.maximum(t + bm1_ref[...], 0.0)
        o = jnp.dot(t, wm2_ref[...], preferred_element_type=jnp.float32)
        o_ref[...] = o + bm2_ref[...]

    return pl.pallas_call(
        body,
        grid=(NRB,),
        in_specs=[
            pl.BlockSpec((RB, DH), lambda i: (i, 0)),
            pl.BlockSpec((RB, DH), lambda i: (i, 0)),
            pl.BlockSpec((RB, DH), lambda i: (i, 0)),
            pl.BlockSpec((RB, DH), lambda i: (i, 0)),
            pl.BlockSpec((RB, 1), lambda i: (i, 0)),
            pl.BlockSpec((1, D), lambda i: (0, 0)),
            pl.BlockSpec((D, DH), lambda i: (0, 0)),
            pl.BlockSpec((1, DH), lambda i: (0, 0)),
            pl.BlockSpec((DH, D), lambda i: (0, 0)),
            pl.BlockSpec((1, D), lambda i: (0, 0)),
        ],
        out_specs=pl.BlockSpec((RB, D), lambda i: (i, 0)),
        out_shape=jax.ShapeDtypeStruct((N, D), jnp.float32),
    )(slo, shi, hplo, hphi, dv, b3, Wm1, bm1, Wm2, bm2)


def kernel(x, edge_index, W1, b1, W2, b2, W3, b3, Wm1, bm1, Wm2, bm2):
    src = edge_index[0].astype(jnp.int32)
    dst = edge_index[1].astype(jnp.int32)
    e = src.shape[0]
    pad = EPS - e
    src3d = jnp.concatenate(
        [src, jnp.zeros((pad,), jnp.int32)]).reshape(NSUB, NCHUNK, CHUNK)
    dst3d = jnp.concatenate(
        [dst, jnp.full((pad,), TRASH, jnp.int32)]).reshape(NSUB, NCHUNK, CHUNK)
    dst2d = dst.reshape(NSUB, e // NSUB)

    dinv = _sc_dinv(dst2d)
    dv = dinv[:N].reshape(N, 1)
    zeros_tab = jnp.zeros((NPAD, DH), jnp.float32)
    b1r = b1.reshape(1, D)
    b2r = b2.reshape(1, D)
    b3r = b3.reshape(1, D)
    bm1r = bm1.reshape(1, DH)
    bm2r = bm2.reshape(1, D)

    h1lo, h1hi = _stage_first(x, W1, dv)
    s1lo, s1hi = _sc_layer(h1lo, h1hi, src3d, dst3d, zeros_tab)
    h2lo, h2hi = _stage_mid(s1lo, s1hi, h1lo, h1hi, dv, b1r, W2)
    s2lo, s2hi = _sc_layer(h2lo, h2hi, src3d, dst3d, zeros_tab)
    h3lo, h3hi = _stage_mid(s2lo, s2hi, h2lo, h2hi, dv, b2r, W3)
    s3lo, s3hi = _sc_layer(h3lo, h3hi, src3d, dst3d, zeros_tab)
    return _stage_final(s3lo, s3hi, h3lo, h3hi, dv, b3r, Wm1, bm1r, Wm2, bm2r)


# speedup vs baseline: 9.2953x; 1.2556x over previous
import dataclasses
import functools

import jax
import jax.numpy as jnp
from jax import lax
from jax.experimental import pallas as pl
from jax.experimental.pallas import tpu as pltpu
from jax.experimental.pallas import tpu_sc as plsc

N = 10000
D = 256
NSUB = 16
NCORE = 2
HALF = N // 2
NPADH = 5248
ROWS_SUB = NPADH // NSUB
TRASH_L = 5240
NPADD = 10240
NODES_TILE = NPADD // (NSUB * NCORE)
CHUNK = 64
NBUF = 2
CPS = 96
SWEEP_E = CPS * CHUNK
PCAP = 2 * SWEEP_E
PBUF = 10368

_mesh = plsc.VectorSubcoreMesh(core_axis_name="c", subcore_axis_name="s")


def _sc_compiler_params():
    cp = pltpu.CompilerParams()
    fields = pltpu.CompilerParams.__dataclass_fields__
    if "needs_layout_passes" in fields:
        cp = dataclasses.replace(cp, needs_layout_passes=False)
    if "use_tc_tiling_on_sc" in fields:
        cp = dataclasses.replace(cp, use_tc_tiling_on_sc=False)
    return cp


def _sc_prep(src2d, dst2d):
    eps = dst2d.shape[1]

    @functools.partial(
        pl.kernel,
        out_type=[jax.ShapeDtypeStruct((NPADD,), jnp.float32),
                  jax.ShapeDtypeStruct((NCORE, NSUB, PCAP), jnp.int32),
                  jax.ShapeDtypeStruct((NCORE, NSUB, PCAP), jnp.int32),
                  jax.ShapeDtypeStruct((NCORE, NSUB, 16), jnp.int32)],
        mesh=_mesh,
        scratch_types=[
            pltpu.VMEM((NPADD,), jnp.float32),
            pltpu.VMEM((eps,), jnp.int32),
            pltpu.VMEM((eps,), jnp.int32),
            pltpu.VMEM((PBUF,), jnp.int32),
            pltpu.VMEM((PBUF,), jnp.int32),
            pltpu.VMEM_SHARED((NSUB, NPADD), jnp.float32),
            pltpu.VMEM((NSUB, NODES_TILE), jnp.float32),
            pltpu.VMEM((NODES_TILE,), jnp.float32),
            pltpu.VMEM((16,), jnp.int32),
        ],
        compiler_params=_sc_compiler_params(),
    )
    def k(src_hbm, dst_hbm, dinv_hbm, psrc_hbm, pdst_hbm, cnt_hbm,
          hist_v, srcs_v, idx_v, psrc_v, pdst_v, part_sh, red_v, dv_v, cnt_v):
        c = lax.axis_index("c")
        s = lax.axis_index("s")
        zero16 = jnp.zeros((16,), jnp.float32)
        ones16 = jnp.full((16,), 1.0, jnp.float32)

        @pl.loop(0, NPADD, step=16)
        def _(i):
            hist_v[pl.ds(i, 16)] = zero16

        pltpu.sync_copy(src_hbm.at[s], srcs_v)
        pltpu.sync_copy(dst_hbm.at[s], idx_v)

        @pl.loop(0, eps, step=16)
        def _(i):
            idx16 = idx_v[pl.ds(i, 16)]
            plsc.addupdate_scatter(hist_v, [idx16], ones16)

        pltpu.sync_copy(hist_v, part_sh.at[s])

        lo = c * HALF

        def body(it, off):
            i = it * 16
            d16 = idx_v[pl.ds(i, 16)] - lo
            s16 = srcs_v[pl.ds(i, 16)]
            m = (d16 >= 0) & (d16 < HALF)
            plsc.store_compressed(psrc_v.at[pl.ds(off, 16)], s16, mask=m)
            plsc.store_compressed(pdst_v.at[pl.ds(off, 16)], d16, mask=m)
            return off + jnp.sum(m.astype(jnp.int32))

        cnt = lax.fori_loop(0, eps // 16, body, jnp.int32(0))
        cpad = ((cnt + (NBUF * CHUNK - 1)) // (NBUF * CHUNK)) * (NBUF * CHUNK)
        zero16i = jnp.zeros((16,), jnp.int32)
        trash16 = jnp.full((16,), TRASH_L, jnp.int32)
        for t in range(NBUF * CHUNK // 16):
            @pl.when(cnt + t * 16 < cpad)
            def _(t=t):
                psrc_v[pl.ds(cnt + t * 16, 16)] = zero16i
                pdst_v[pl.ds(cnt + t * 16, 16)] = trash16

        cnt_v[...] = jnp.full((16,), cpad // CHUNK, jnp.int32)
        pltpu.sync_copy(cnt_v, cnt_hbm.at[c, s])
        pltpu.sync_copy(psrc_v, psrc_hbm.at[c, s, pl.ds(0, PBUF)])
        pltpu.sync_copy(pdst_v, pdst_hbm.at[c, s, pl.ds(0, PBUF)])

        plsc.subcore_barrier()

        base = (c * NSUB + s) * NODES_TILE
        for r in range(NSUB):
            pltpu.sync_copy(part_sh.at[r, pl.ds(base, NODES_TILE)],
                            red_v.at[r])

        @pl.loop(0, NODES_TILE, step=16)
        def _(j):
            acc = jnp.full((16,), 1.0, jnp.float32)
            for r in range(NSUB):
                acc = acc + red_v[r, pl.ds(j, 16)]
            bits = plsc.bitcast(acc, jnp.int32)
            y = plsc.bitcast(jnp.int32(0x5F3759DF) - (bits >> 1), jnp.float32)
            for _ in range(3):
                y = y * (1.5 - 0.5 * acc * y * y)
            dv_v[pl.ds(j, 16)] = y

        pltpu.sync_copy(dv_v, dinv_hbm.at[pl.ds(base, NODES_TILE)])

    return k(src2d, dst2d)


def _sc_layer(h, psrc, pdst, cnts, zeros_tab):
    @functools.partial(
        pl.kernel,
        out_type=jax.ShapeDtypeStruct((NCORE, NPADH, D), jnp.float32),
        mesh=_mesh,
        scratch_types=(
            [pltpu.VMEM((SWEEP_E,), jnp.int32),
             pltpu.VMEM((SWEEP_E,), jnp.int32)]
            + [pltpu.VMEM((CHUNK, D), jnp.float32) for _ in range(NBUF)]
            + [pltpu.VMEM_SHARED((NPADH, D), jnp.float32)]
            + [pltpu.VMEM_SHARED((NSUB, 16), jnp.int32)]
            + [pltpu.SMEM((16,), jnp.int32)]
            + [pltpu.SemaphoreType.DMA for _ in range(2 * NBUF)]
        ),
        compiler_params=_sc_compiler_params(),
    )
    def k(h_hbm, psrc_hbm, pdst_hbm, cnt_hbm, z_hbm, out_hbm,
          src_v, dst_v, *rest):
        rows = rest[:NBUF]
        acc_sh = rest[NBUF]
        cnt_v = rest[NBUF + 1]
        cnt_sm = rest[NBUF + 2]
        gsem = rest[NBUF + 3:NBUF + 3 + NBUF]
        ssem = rest[NBUF + 3 + NBUF:]
        c = lax.axis_index("c")
        s = lax.axis_index("s")
        row0 = s * ROWS_SUB

        pltpu.sync_copy(cnt_hbm.at[c, s], cnt_v.at[s])
        pltpu.sync_copy(cnt_v.at[s], cnt_sm)
        pltpu.sync_copy(z_hbm.at[pl.ds(row0, ROWS_SUB)],
                        acc_sh.at[pl.ds(row0, ROWS_SUB)])
        plsc.subcore_barrier()

        nch = cnt_sm[0]

        def gstart(kk, b):
            pltpu.async_copy(
                h_hbm.at[src_v.at[pl.ds(kk * CHUNK, CHUNK)]], rows[b], gsem[b])

        def gwait(kk, b):
            pltpu.make_async_copy(
                h_hbm.at[src_v.at[pl.ds(kk * CHUNK, CHUNK)]], rows[b],
                gsem[b]).wait()

        def sstart(kk, b):
            pltpu.async_copy(
                rows[b], acc_sh.at[dst_v.at[pl.ds(kk * CHUNK, CHUNK)]],
                ssem[b], add=True)

        def swait(kk, b):
            pltpu.make_async_copy(
                rows[b], acc_sh.at[dst_v.at[pl.ds(kk * CHUNK, CHUNK)]],
                ssem[b]).wait()

        @pl.loop(0, (nch + CPS - 1) // CPS)
        def _(w):
            pltpu.sync_copy(psrc_hbm.at[c, s, pl.ds(w * SWEEP_E, SWEEP_E)],
                            src_v)
            pltpu.sync_copy(pdst_hbm.at[c, s, pl.ds(w * SWEEP_E, SWEEP_E)],
                            dst_v)
            cps = jnp.minimum(jnp.int32(CPS), nch - w * CPS)
            gstart(0, 0)
            gstart(1, 1)

            @pl.loop(0, cps, step=2)
            def _(kk):
                gwait(kk, 0)
                sstart(kk, 0)
                gwait(kk + 1, 1)
                sstart(kk + 1, 1)

                @pl.when(kk + 2 < cps)
                def _():
                    swait(kk, 0)
                    gstart(kk + 2, 0)
                    swait(kk + 1, 1)
                    gstart(kk + 3, 1)

            swait(cps - 2, 0)
            swait(cps - 1, 1)

        plsc.subcore_barrier()
        pltpu.sync_copy(acc_sh.at[pl.ds(row0, ROWS_SUB)],
                        out_hbm.at[c, pl.ds(row0, ROWS_SUB)])

    return k(h, psrc, pdst, cnts, zeros_tab)


RB = 1000
NRB = N // RB
BPC = HALF // RB


def _s_spec():
    return pl.BlockSpec((1, RB, D), lambda i: (i // BPC, i % BPC, 0))


def _stage_first(x, W1, dv):
    def body(x_ref, w_ref, dv_ref, o_ref):
        h = jnp.dot(x_ref[...], w_ref[...], preferred_element_type=jnp.float32)
        o_ref[...] = h * dv_ref[...]

    return pl.pallas_call(
        body,
        grid=(NRB,),
        in_specs=[
            pl.BlockSpec((RB, D), lambda i: (i, 0)),
            pl.BlockSpec((D, D), lambda i: (0, 0)),
            pl.BlockSpec((RB, 1), lambda i: (i, 0)),
        ],
        out_specs=pl.BlockSpec((RB, D), lambda i: (i, 0)),
        out_shape=jax.ShapeDtypeStruct((N, D), jnp.float32),
    )(x, W1, dv)


def _stage_mid(S, hp, dv, b, W):
    def body(s_ref, hp_ref, dv_ref, b_ref, w_ref, o_ref):
        dvb = dv_ref[...]
        conv = (s_ref[0] + hp_ref[...]) * dvb + b_ref[...]
        hh = jnp.maximum(conv, 0.0)
        o_ref[...] = jnp.dot(hh, w_ref[...],
                             preferred_element_type=jnp.float32) * dvb

    return pl.pallas_call(
        body,
        grid=(NRB,),
        in_specs=[
            _s_spec(),
            pl.BlockSpec((RB, D), lambda i: (i, 0)),
            pl.BlockSpec((RB, 1), lambda i: (i, 0)),
            pl.BlockSpec((1, D), lambda i: (0, 0)),
            pl.BlockSpec((D, D), lambda i: (0, 0)),
        ],
        out_specs=pl.BlockSpec((RB, D), lambda i: (i, 0)),
        out_shape=jax.ShapeDtypeStruct((N, D), jnp.float32),
    )(S, hp, dv, b, W)


def _stage_final(S, hp, dv, b3, Wm1, bm1, Wm2, bm2):
    def body(s_ref, hp_ref, dv_ref, b3_ref, wm1_ref, bm1_ref, wm2_ref,
             bm2_ref, o_ref):
        conv = (s_ref[0] + hp_ref[...]) * dv_ref[...] + b3_ref[...]
        t = jnp.dot(conv, wm1_ref[...], preferred_element_type=jnp.float32)
        t = jnp.maximum(t + bm1_ref[...], 0.0)
        o = jnp.dot(t, wm2_ref[...], preferred_element_type=jnp.float32)
        o_ref[...] = o + bm2_ref[...]

    return pl.pallas_call(
        body,
        grid=(NRB,),
        in_specs=[
            _s_spec(),
            pl.BlockSpec((RB, D), lambda i: (i, 0)),
            pl.BlockSpec((RB, 1), lambda i: (i, 0)),
            pl.BlockSpec((1, D), lambda i: (0, 0)),
            pl.BlockSpec((D, D // 2), lambda i: (0, 0)),
            pl.BlockSpec((1, D // 2), lambda i: (0, 0)),
            pl.BlockSpec((D // 2, D), lambda i: (0, 0)),
            pl.BlockSpec((1, D), lambda i: (0, 0)),
        ],
        out_specs=pl.BlockSpec((RB, D), lambda i: (i, 0)),
        out_shape=jax.ShapeDtypeStruct((N, D), jnp.float32),
    )(S, hp, dv, b3, Wm1, bm1, Wm2, bm2)


def kernel(x, edge_index, W1, b1, W2, b2, W3, b3, Wm1, bm1, Wm2, bm2):
    src = edge_index[0].astype(jnp.int32)
    dst = edge_index[1].astype(jnp.int32)
    e = src.shape[0]
    src2d = src.reshape(NSUB, e // NSUB)
    dst2d = dst.reshape(NSUB, e // NSUB)

    dinv, psrc, pdst, cnts = _sc_prep(src2d, dst2d)
    dv = dinv[:N].reshape(N, 1)
    zeros_tab = jnp.zeros((NPADH, D), jnp.float32)
    b1r = b1.reshape(1, D)
    b2r = b2.reshape(1, D)
    b3r = b3.reshape(1, D)
    bm1r = bm1.reshape(1, D // 2)
    bm2r = bm2.reshape(1, D)

    h1 = _stage_first(x, W1, dv)
    s1 = _sc_layer(h1, psrc, pdst, cnts, zeros_tab)
    h2 = _stage_mid(s1, h1, dv, b1r, W2)
    s2 = _sc_layer(h2, psrc, pdst, cnts, zeros_tab)
    h3 = _stage_mid(s2, h2, dv, b2r, W3)
    s3 = _sc_layer(h3, psrc, pdst, cnts, zeros_tab)
    return _stage_final(s3, h3, dv, b3r, Wm1, bm1r, Wm2, bm2r)


# speedup vs baseline: 10.7661x; 1.1582x over previous
import dataclasses
import functools

import jax
import jax.numpy as jnp
from jax import lax
from jax.experimental import pallas as pl
from jax.experimental.pallas import tpu as pltpu
from jax.experimental.pallas import tpu_sc as plsc

N = 10000
D = 256
NSUB = 16
NCORE = 2
HALF = N // 2
NPADH = 5248
ROWS_SUB = NPADH // NSUB
TRASH_L = 5240
NPADD = 10240
NODES_TILE = NPADD // (NSUB * NCORE)
CHUNK = 32
NBUF = 4
CPS = 192
SWEEP_E = CPS * CHUNK
PCAP = 2 * SWEEP_E
PBUF = 10368

_mesh = plsc.VectorSubcoreMesh(core_axis_name="c", subcore_axis_name="s")


def _sc_compiler_params():
    cp = pltpu.CompilerParams()
    fields = pltpu.CompilerParams.__dataclass_fields__
    if "needs_layout_passes" in fields:
        cp = dataclasses.replace(cp, needs_layout_passes=False)
    if "use_tc_tiling_on_sc" in fields:
        cp = dataclasses.replace(cp, use_tc_tiling_on_sc=False)
    return cp


def _sc_prep(src2d, dst2d):
    eps = dst2d.shape[1]

    @functools.partial(
        pl.kernel,
        out_type=[jax.ShapeDtypeStruct((NPADD,), jnp.float32),
                  jax.ShapeDtypeStruct((NCORE, NSUB, PCAP), jnp.int32),
                  jax.ShapeDtypeStruct((NCORE, NSUB, PCAP), jnp.int32),
                  jax.ShapeDtypeStruct((NCORE, NSUB, 16), jnp.int32)],
        mesh=_mesh,
        scratch_types=[
            pltpu.VMEM((NPADD,), jnp.float32),
            pltpu.VMEM((eps,), jnp.int32),
            pltpu.VMEM((eps,), jnp.int32),
            pltpu.VMEM((PBUF,), jnp.int32),
            pltpu.VMEM((PBUF,), jnp.int32),
            pltpu.VMEM_SHARED((NSUB, NPADD), jnp.float32),
            pltpu.VMEM((NSUB, NODES_TILE), jnp.float32),
            pltpu.VMEM((NODES_TILE,), jnp.float32),
            pltpu.VMEM((16,), jnp.int32),
        ],
        compiler_params=_sc_compiler_params(),
    )
    def k(src_hbm, dst_hbm, dinv_hbm, psrc_hbm, pdst_hbm, cnt_hbm,
          hist_v, srcs_v, idx_v, psrc_v, pdst_v, part_sh, red_v, dv_v, cnt_v):
        c = lax.axis_index("c")
        s = lax.axis_index("s")
        zero16 = jnp.zeros((16,), jnp.float32)
        ones16 = jnp.full((16,), 1.0, jnp.float32)

        @pl.loop(0, NPADD, step=16)
        def _(i):
            hist_v[pl.ds(i, 16)] = zero16

        pltpu.sync_copy(src_hbm.at[s], srcs_v)
        pltpu.sync_copy(dst_hbm.at[s], idx_v)

        @pl.loop(0, eps, step=16)
        def _(i):
            idx16 = idx_v[pl.ds(i, 16)]
            plsc.addupdate_scatter(hist_v, [idx16], ones16)

        pltpu.sync_copy(hist_v, part_sh.at[s])

        lo = c * HALF

        def body(it, off):
            i = it * 16
            d16 = idx_v[pl.ds(i, 16)] - lo
            s16 = srcs_v[pl.ds(i, 16)]
            m = (d16 >= 0) & (d16 < HALF)
            plsc.store_compressed(psrc_v.at[pl.ds(off, 16)], s16, mask=m)
            plsc.store_compressed(pdst_v.at[pl.ds(off, 16)], d16, mask=m)
            return off + jnp.sum(m.astype(jnp.int32))

        cnt = lax.fori_loop(0, eps // 16, body, jnp.int32(0))
        cpad = ((cnt + (NBUF * CHUNK - 1)) // (NBUF * CHUNK)) * (NBUF * CHUNK)
        zero16i = jnp.zeros((16,), jnp.int32)
        trash16 = jnp.full((16,), TRASH_L, jnp.int32)
        for t in range(NBUF * CHUNK // 16):
            @pl.when(cnt + t * 16 < cpad)
            def _(t=t):
                psrc_v[pl.ds(cnt + t * 16, 16)] = zero16i
                pdst_v[pl.ds(cnt + t * 16, 16)] = trash16

        cnt_v[...] = jnp.full((16,), cpad // CHUNK, jnp.int32)
        pltpu.sync_copy(cnt_v, cnt_hbm.at[c, s])
        pltpu.sync_copy(psrc_v, psrc_hbm.at[c, s, pl.ds(0, PBUF)])
        pltpu.sync_copy(pdst_v, pdst_hbm.at[c, s, pl.ds(0, PBUF)])

        plsc.subcore_barrier()

        base = (c * NSUB + s) * NODES_TILE
        for r in range(NSUB):
            pltpu.sync_copy(part_sh.at[r, pl.ds(base, NODES_TILE)],
                            red_v.at[r])

        @pl.loop(0, NODES_TILE, step=16)
        def _(j):
            acc = jnp.full((16,), 1.0, jnp.float32)
            for r in range(NSUB):
                acc = acc + red_v[r, pl.ds(j, 16)]
            bits = plsc.bitcast(acc, jnp.int32)
            y = plsc.bitcast(jnp.int32(0x5F3759DF) - (bits >> 1), jnp.float32)
            for _ in range(3):
                y = y * (1.5 - 0.5 * acc * y * y)
            dv_v[pl.ds(j, 16)] = y

        pltpu.sync_copy(dv_v, dinv_hbm.at[pl.ds(base, NODES_TILE)])

    return k(src2d, dst2d)


def _sc_layer(h, psrc, pdst, cnts, zeros_tab):
    @functools.partial(
        pl.kernel,
        out_type=jax.ShapeDtypeStruct((NCORE, NPADH, D), jnp.float32),
        mesh=_mesh,
        scratch_types=(
            [pltpu.VMEM((SWEEP_E,), jnp.int32),
             pltpu.VMEM((SWEEP_E,), jnp.int32)]
            + [pltpu.VMEM((CHUNK, D), jnp.float32) for _ in range(NBUF)]
            + [pltpu.VMEM_SHARED((NPADH, D), jnp.float32)]
            + [pltpu.VMEM_SHARED((NSUB, 16), jnp.int32)]
            + [pltpu.SMEM((16,), jnp.int32)]
            + [pltpu.SemaphoreType.DMA for _ in range(2 * NBUF)]
        ),
        compiler_params=_sc_compiler_params(),
    )
    def k(h_hbm, psrc_hbm, pdst_hbm, cnt_hbm, z_hbm, out_hbm,
          src_v, dst_v, *rest):
        rows = rest[:NBUF]
        acc_sh = rest[NBUF]
        cnt_v = rest[NBUF + 1]
        cnt_sm = rest[NBUF + 2]
        gsem = rest[NBUF + 3:NBUF + 3 + NBUF]
        ssem = rest[NBUF + 3 + NBUF:]
        c = lax.axis_index("c")
        s = lax.axis_index("s")
        row0 = s * ROWS_SUB

        pltpu.sync_copy(cnt_hbm.at[c, s], cnt_v.at[s])
        pltpu.sync_copy(cnt_v.at[s], cnt_sm)
        pltpu.sync_copy(z_hbm, acc_sh.at[pl.ds(row0, ROWS_SUB)])
        plsc.subcore_barrier()

        nch = cnt_sm[0]

        def gstart(kk, b):
            pltpu.async_copy(
                h_hbm.at[src_v.at[pl.ds(kk * CHUNK, CHUNK)]], rows[b], gsem[b])

        def gwait(kk, b):
            pltpu.make_async_copy(
                h_hbm.at[src_v.at[pl.ds(kk * CHUNK, CHUNK)]], rows[b],
                gsem[b]).wait()

        def sstart(kk, b):
            pltpu.async_copy(
                rows[b], acc_sh.at[dst_v.at[pl.ds(kk * CHUNK, CHUNK)]],
                ssem[b], add=True)

        def swait(kk, b):
            pltpu.make_async_copy(
                rows[b], acc_sh.at[dst_v.at[pl.ds(kk * CHUNK, CHUNK)]],
                ssem[b]).wait()

        @pl.loop(0, (nch + CPS - 1) // CPS)
        def _(w):
            pltpu.sync_copy(psrc_hbm.at[c, s, pl.ds(w * SWEEP_E, SWEEP_E)],
                            src_v)
            pltpu.sync_copy(pdst_hbm.at[c, s, pl.ds(w * SWEEP_E, SWEEP_E)],
                            dst_v)
            cps = jnp.minimum(jnp.int32(CPS), nch - w * CPS)
            for r in range(NBUF - 1):
                gstart(r, r)

            @pl.loop(0, cps, step=NBUF)
            def _(kk):
                for r in range(NBUF):
                    j = kk + r
                    bn = (r + NBUF - 1) % NBUF
                    gwait(j, r)
                    sstart(j, r)

                    @pl.when(j + NBUF - 1 < cps)
                    def _(j=j, bn=bn):
                        @pl.when(j >= 1)
                        def _():
                            swait(j - 1, bn)

                        gstart(j + NBUF - 1, bn)

            for r in range(NBUF):
                swait(cps - NBUF + r, r)

        plsc.subcore_barrier()
        pltpu.sync_copy(acc_sh.at[pl.ds(row0, ROWS_SUB)],
                        out_hbm.at[c, pl.ds(row0, ROWS_SUB)])

    return k(h, psrc, pdst, cnts, zeros_tab)


RB = 1000
NRB = N // RB
BPC = HALF // RB


def _s_spec():
    return pl.BlockSpec((1, RB, D), lambda i: (i // BPC, i % BPC, 0))


def _stage_first(x, W1, dv):
    def body(x_ref, w_ref, dv_ref, o_ref):
        h = jnp.dot(x_ref[...], w_ref[...], preferred_element_type=jnp.float32)
        o_ref[...] = h * dv_ref[...]

    return pl.pallas_call(
        body,
        grid=(NRB,),
        in_specs=[
            pl.BlockSpec((RB, D), lambda i: (i, 0)),
            pl.BlockSpec((D, D), lambda i: (0, 0)),
            pl.BlockSpec((RB, 1), lambda i: (i, 0)),
        ],
        out_specs=pl.BlockSpec((RB, D), lambda i: (i, 0)),
        out_shape=jax.ShapeDtypeStruct((N, D), jnp.float32),
    )(x, W1, dv)


def _stage_mid(S, hp, dv, b, W):
    def body(s_ref, hp_ref, dv_ref, b_ref, w_ref, o_ref):
        dvb = dv_ref[...]
        conv = (s_ref[0] + hp_ref[...]) * dvb + b_ref[...]
        hh = jnp.maximum(conv, 0.0)
        o_ref[...] = jnp.dot(hh, w_ref[...],
                             preferred_element_type=jnp.float32) * dvb

    return pl.pallas_call(
        body,
        grid=(NRB,),
        in_specs=[
            _s_spec(),
            pl.BlockSpec((RB, D), lambda i: (i, 0)),
            pl.BlockSpec((RB, 1), lambda i: (i, 0)),
            pl.BlockSpec((1, D), lambda i: (0, 0)),
            pl.BlockSpec((D, D), lambda i: (0, 0)),
        ],
        out_specs=pl.BlockSpec((RB, D), lambda i: (i, 0)),
        out_shape=jax.ShapeDtypeStruct((N, D), jnp.float32),
    )(S, hp, dv, b, W)


def _stage_final(S, hp, dv, b3, Wm1, bm1, Wm2, bm2):
    def body(s_ref, hp_ref, dv_ref, b3_ref, wm1_ref, bm1_ref, wm2_ref,
             bm2_ref, o_ref):
        conv = (s_ref[0] + hp_ref[...]) * dv_ref[...] + b3_ref[...]
        t = jnp.dot(conv, wm1_ref[...], preferred_element_type=jnp.float32)
        t = jnp.maximum(t + bm1_ref[...], 0.0)
        o = jnp.dot(t, wm2_ref[...], preferred_element_type=jnp.float32)
        o_ref[...] = o + bm2_ref[...]

    return pl.pallas_call(
        body,
        grid=(NRB,),
        in_specs=[
            _s_spec(),
            pl.BlockSpec((RB, D), lambda i: (i, 0)),
            pl.BlockSpec((RB, 1), lambda i: (i, 0)),
            pl.BlockSpec((1, D), lambda i: (0, 0)),
            pl.BlockSpec((D, D // 2), lambda i: (0, 0)),
            pl.BlockSpec((1, D // 2), lambda i: (0, 0)),
            pl.BlockSpec((D // 2, D), lambda i: (0, 0)),
            pl.BlockSpec((1, D), lambda i: (0, 0)),
        ],
        out_specs=pl.BlockSpec((RB, D), lambda i: (i, 0)),
        out_shape=jax.ShapeDtypeStruct((N, D), jnp.float32),
    )(S, hp, dv, b3, Wm1, bm1, Wm2, bm2)


def kernel(x, edge_index, W1, b1, W2, b2, W3, b3, Wm1, bm1, Wm2, bm2):
    src = edge_index[0].astype(jnp.int32)
    dst = edge_index[1].astype(jnp.int32)
    e = src.shape[0]
    src2d = src.reshape(NSUB, e // NSUB)
    dst2d = dst.reshape(NSUB, e // NSUB)

    dinv, psrc, pdst, cnts = _sc_prep(src2d, dst2d)
    dv = dinv[:N].reshape(N, 1)
    zeros_tab = jnp.zeros((ROWS_SUB, D), jnp.float32)
    b1r = b1.reshape(1, D)
    b2r = b2.reshape(1, D)
    b3r = b3.reshape(1, D)
    bm1r = bm1.reshape(1, D // 2)
    bm2r = bm2.reshape(1, D)

    h1 = _stage_first(x, W1, dv)
    s1 = _sc_layer(h1, psrc, pdst, cnts, zeros_tab)
    h2 = _stage_mid(s1, h1, dv, b1r, W2)
    s2 = _sc_layer(h2, psrc, pdst, cnts, zeros_tab)
    h3 = _stage_mid(s2, h2, dv, b2r, W3)
    s3 = _sc_layer(h3, psrc, pdst, cnts, zeros_tab)
    return _stage_final(s3, h3, dv, b3r, Wm1, bm1r, Wm2, bm2r)


# speedup vs baseline: 11.0854x; 1.0297x over previous
import dataclasses
import functools

import jax
import jax.numpy as jnp
from jax import lax
from jax.experimental import pallas as pl
from jax.experimental.pallas import tpu as pltpu
from jax.experimental.pallas import tpu_sc as plsc

N = 10000
D = 256
NSUB = 16
NCORE = 2
HALF = N // 2
NPADH = 5248
ROWS_SUB = NPADH // NSUB
TRASH_L = 5240
NPADD = 10240
NODES_TILE = NPADD // (NSUB * NCORE)
CHUNK = 16
NBUF = 8
CPS = 384
SWEEP_E = CPS * CHUNK
PCAP = 2 * SWEEP_E
PBUF = 10368

_mesh = plsc.VectorSubcoreMesh(core_axis_name="c", subcore_axis_name="s")


def _sc_compiler_params():
    cp = pltpu.CompilerParams()
    fields = pltpu.CompilerParams.__dataclass_fields__
    if "needs_layout_passes" in fields:
        cp = dataclasses.replace(cp, needs_layout_passes=False)
    if "use_tc_tiling_on_sc" in fields:
        cp = dataclasses.replace(cp, use_tc_tiling_on_sc=False)
    return cp


def _sc_prep(src2d, dst2d):
    eps = dst2d.shape[1]

    @functools.partial(
        pl.kernel,
        out_type=[jax.ShapeDtypeStruct((NPADD,), jnp.float32),
                  jax.ShapeDtypeStruct((NCORE, NSUB, PCAP), jnp.int32),
                  jax.ShapeDtypeStruct((NCORE, NSUB, PCAP), jnp.int32),
                  jax.ShapeDtypeStruct((NCORE, NSUB, 16), jnp.int32)],
        mesh=_mesh,
        scratch_types=[
            pltpu.VMEM((NPADD,), jnp.float32),
            pltpu.VMEM((eps,), jnp.int32),
            pltpu.VMEM((eps,), jnp.int32),
            pltpu.VMEM((PBUF,), jnp.int32),
            pltpu.VMEM((PBUF,), jnp.int32),
            pltpu.VMEM_SHARED((NSUB, NPADD), jnp.float32),
            pltpu.VMEM((NSUB, NODES_TILE), jnp.float32),
            pltpu.VMEM((NODES_TILE,), jnp.float32),
            pltpu.VMEM((16,), jnp.int32),
        ],
        compiler_params=_sc_compiler_params(),
    )
    def k(src_hbm, dst_hbm, dinv_hbm, psrc_hbm, pdst_hbm, cnt_hbm,
          hist_v, srcs_v, idx_v, psrc_v, pdst_v, part_sh, red_v, dv_v, cnt_v):
        c = lax.axis_index("c")
        s = lax.axis_index("s")
        zero16 = jnp.zeros((16,), jnp.float32)
        ones16 = jnp.full((16,), 1.0, jnp.float32)

        @pl.loop(0, NPADD, step=16)
        def _(i):
            hist_v[pl.ds(i, 16)] = zero16

        pltpu.sync_copy(src_hbm.at[s], srcs_v)
        pltpu.sync_copy(dst_hbm.at[s], idx_v)

        @pl.loop(0, eps, step=16)
        def _(i):
            idx16 = idx_v[pl.ds(i, 16)]
            plsc.addupdate_scatter(hist_v, [idx16], ones16)

        pltpu.sync_copy(hist_v, part_sh.at[s])

        lo = c * HALF

        def body(it, off):
            i = it * 16
            d16 = idx_v[pl.ds(i, 16)] - lo
            s16 = srcs_v[pl.ds(i, 16)]
            m = (d16 >= 0) & (d16 < HALF)
            plsc.store_compressed(psrc_v.at[pl.ds(off, 16)], s16, mask=m)
            plsc.store_compressed(pdst_v.at[pl.ds(off, 16)], d16, mask=m)
            return off + jnp.sum(m.astype(jnp.int32))

        cnt = lax.fori_loop(0, eps // 16, body, jnp.int32(0))
        cpad = ((cnt + (NBUF * CHUNK - 1)) // (NBUF * CHUNK)) * (NBUF * CHUNK)
        zero16i = jnp.zeros((16,), jnp.int32)
        trash16 = jnp.full((16,), TRASH_L, jnp.int32)
        for t in range(NBUF * CHUNK // 16):
            @pl.when(cnt + t * 16 < cpad)
            def _(t=t):
                psrc_v[pl.ds(cnt + t * 16, 16)] = zero16i
                pdst_v[pl.ds(cnt + t * 16, 16)] = trash16

        cnt_v[...] = jnp.full((16,), cpad // CHUNK, jnp.int32)
        pltpu.sync_copy(cnt_v, cnt_hbm.at[c, s])
        pltpu.sync_copy(psrc_v, psrc_hbm.at[c, s, pl.ds(0, PBUF)])
        pltpu.sync_copy(pdst_v, pdst_hbm.at[c, s, pl.ds(0, PBUF)])

        plsc.subcore_barrier()

        base = (c * NSUB + s) * NODES_TILE
        for r in range(NSUB):
            pltpu.sync_copy(part_sh.at[r, pl.ds(base, NODES_TILE)],
                            red_v.at[r])

        @pl.loop(0, NODES_TILE, step=16)
        def _(j):
            acc = jnp.full((16,), 1.0, jnp.float32)
            for r in range(NSUB):
                acc = acc + red_v[r, pl.ds(j, 16)]
            bits = plsc.bitcast(acc, jnp.int32)
            y = plsc.bitcast(jnp.int32(0x5F3759DF) - (bits >> 1), jnp.float32)
            for _ in range(3):
                y = y * (1.5 - 0.5 * acc * y * y)
            dv_v[pl.ds(j, 16)] = y

        pltpu.sync_copy(dv_v, dinv_hbm.at[pl.ds(base, NODES_TILE)])

    return k(src2d, dst2d)


def _sc_layer(h, psrc, pdst, cnts, zeros_tab):
    @functools.partial(
        pl.kernel,
        out_type=jax.ShapeDtypeStruct((NCORE, NPADH, D), jnp.float32),
        mesh=_mesh,
        scratch_types=(
            [pltpu.VMEM((SWEEP_E,), jnp.int32),
             pltpu.VMEM((SWEEP_E,), jnp.int32)]
            + [pltpu.VMEM((CHUNK, D), jnp.float32) for _ in range(NBUF)]
            + [pltpu.VMEM_SHARED((NPADH, D), jnp.float32)]
            + [pltpu.VMEM_SHARED((NSUB, 16), jnp.int32)]
            + [pltpu.SMEM((16,), jnp.int32)]
            + [pltpu.SemaphoreType.DMA for _ in range(2 * NBUF)]
        ),
        compiler_params=_sc_compiler_params(),
    )
    def k(h_hbm, psrc_hbm, pdst_hbm, cnt_hbm, z_hbm, out_hbm,
          src_v, dst_v, *rest):
        rows = rest[:NBUF]
        acc_sh = rest[NBUF]
        cnt_v = rest[NBUF + 1]
        cnt_sm = rest[NBUF + 2]
        gsem = rest[NBUF + 3:NBUF + 3 + NBUF]
        ssem = rest[NBUF + 3 + NBUF:]
        c = lax.axis_index("c")
        s = lax.axis_index("s")
        row0 = s * ROWS_SUB

        pltpu.sync_copy(cnt_hbm.at[c, s], cnt_v.at[s])
        pltpu.sync_copy(cnt_v.at[s], cnt_sm)
        pltpu.sync_copy(z_hbm, acc_sh.at[pl.ds(row0, ROWS_SUB)])
        plsc.subcore_barrier()

        nch = cnt_sm[0]

        def gstart(kk, b):
            pltpu.async_copy(
                h_hbm.at[src_v.at[pl.ds(kk * CHUNK, CHUNK)]], rows[b], gsem[b])

        def gwait(kk, b):
            pltpu.make_async_copy(
                h_hbm.at[src_v.at[pl.ds(kk * CHUNK, CHUNK)]], rows[b],
                gsem[b]).wait()

        def sstart(kk, b):
            pltpu.async_copy(
                rows[b], acc_sh.at[dst_v.at[pl.ds(kk * CHUNK, CHUNK)]],
                ssem[b], add=True)

        def swait(kk, b):
            pltpu.make_async_copy(
                rows[b], acc_sh.at[dst_v.at[pl.ds(kk * CHUNK, CHUNK)]],
                ssem[b]).wait()

        @pl.loop(0, (nch + CPS - 1) // CPS)
        def _(w):
            pltpu.sync_copy(psrc_hbm.at[c, s, pl.ds(w * SWEEP_E, SWEEP_E)],
                            src_v)
            pltpu.sync_copy(pdst_hbm.at[c, s, pl.ds(w * SWEEP_E, SWEEP_E)],
                            dst_v)
            cps = jnp.minimum(jnp.int32(CPS), nch - w * CPS)
            for r in range(NBUF - 1):
                gstart(r, r)

            @pl.loop(0, cps, step=NBUF)
            def _(kk):
                for r in range(NBUF):
                    j = kk + r
                    bn = (r + NBUF - 1) % NBUF
                    gwait(j, r)
                    sstart(j, r)

                    @pl.when(j + NBUF - 1 < cps)
                    def _(j=j, bn=bn):
                        @pl.when(j >= 1)
                        def _():
                            swait(j - 1, bn)

                        gstart(j + NBUF - 1, bn)

            for r in range(NBUF):
                swait(cps - NBUF + r, r)

        plsc.subcore_barrier()
        pltpu.sync_copy(acc_sh.at[pl.ds(row0, ROWS_SUB)],
                        out_hbm.at[c, pl.ds(row0, ROWS_SUB)])

    return k(h, psrc, pdst, cnts, zeros_tab)


RB = 1000
NRB = N // RB
BPC = HALF // RB


def _s_spec():
    return pl.BlockSpec((1, RB, D), lambda i: (i // BPC, i % BPC, 0))


def _stage_first(x, W1, dv):
    def body(x_ref, w_ref, dv_ref, o_ref):
        h = jnp.dot(x_ref[...], w_ref[...], preferred_element_type=jnp.float32)
        o_ref[...] = h * dv_ref[...]

    return pl.pallas_call(
        body,
        grid=(NRB,),
        in_specs=[
            pl.BlockSpec((RB, D), lambda i: (i, 0)),
            pl.BlockSpec((D, D), lambda i: (0, 0)),
            pl.BlockSpec((RB, 1), lambda i: (i, 0)),
        ],
        out_specs=pl.BlockSpec((RB, D), lambda i: (i, 0)),
        out_shape=jax.ShapeDtypeStruct((N, D), jnp.float32),
    )(x, W1, dv)


def _stage_mid(S, hp, dv, b, W):
    def body(s_ref, hp_ref, dv_ref, b_ref, w_ref, o_ref):
        dvb = dv_ref[...]
        conv = (s_ref[0] + hp_ref[...]) * dvb + b_ref[...]
        hh = jnp.maximum(conv, 0.0)
        o_ref[...] = jnp.dot(hh, w_ref[...],
                             preferred_element_type=jnp.float32) * dvb

    return pl.pallas_call(
        body,
        grid=(NRB,),
        in_specs=[
            _s_spec(),
            pl.BlockSpec((RB, D), lambda i: (i, 0)),
            pl.BlockSpec((RB, 1), lambda i: (i, 0)),
            pl.BlockSpec((1, D), lambda i: (0, 0)),
            pl.BlockSpec((D, D), lambda i: (0, 0)),
        ],
        out_specs=pl.BlockSpec((RB, D), lambda i: (i, 0)),
        out_shape=jax.ShapeDtypeStruct((N, D), jnp.float32),
    )(S, hp, dv, b, W)


def _stage_final(S, hp, dv, b3, Wm1, bm1, Wm2, bm2):
    def body(s_ref, hp_ref, dv_ref, b3_ref, wm1_ref, bm1_ref, wm2_ref,
             bm2_ref, o_ref):
        conv = (s_ref[0] + hp_ref[...]) * dv_ref[...] + b3_ref[...]
        t = jnp.dot(conv, wm1_ref[...], preferred_element_type=jnp.float32)
        t = jnp.maximum(t + bm1_ref[...], 0.0)
        o = jnp.dot(t, wm2_ref[...], preferred_element_type=jnp.float32)
        o_ref[...] = o + bm2_ref[...]

    return pl.pallas_call(
        body,
        grid=(NRB,),
        in_specs=[
            _s_spec(),
            pl.BlockSpec((RB, D), lambda i: (i, 0)),
            pl.BlockSpec((RB, 1), lambda i: (i, 0)),
            pl.BlockSpec((1, D), lambda i: (0, 0)),
            pl.BlockSpec((D, D // 2), lambda i: (0, 0)),
            pl.BlockSpec((1, D // 2), lambda i: (0, 0)),
            pl.BlockSpec((D // 2, D), lambda i: (0, 0)),
            pl.BlockSpec((1, D), lambda i: (0, 0)),
        ],
        out_specs=pl.BlockSpec((RB, D), lambda i: (i, 0)),
        out_shape=jax.ShapeDtypeStruct((N, D), jnp.float32),
    )(S, hp, dv, b3, Wm1, bm1, Wm2, bm2)


def kernel(x, edge_index, W1, b1, W2, b2, W3, b3, Wm1, bm1, Wm2, bm2):
    src = edge_index[0].astype(jnp.int32)
    dst = edge_index[1].astype(jnp.int32)
    e = src.shape[0]
    src2d = src.reshape(NSUB, e // NSUB)
    dst2d = dst.reshape(NSUB, e // NSUB)

    dinv, psrc, pdst, cnts = _sc_prep(src2d, dst2d)
    dv = dinv[:N].reshape(N, 1)
    zeros_tab = jnp.zeros((ROWS_SUB, D), jnp.float32)
    b1r = b1.reshape(1, D)
    b2r = b2.reshape(1, D)
    b3r = b3.reshape(1, D)
    bm1r = bm1.reshape(1, D // 2)
    bm2r = bm2.reshape(1, D)

    h1 = _stage_first(x, W1, dv)
    s1 = _sc_layer(h1, psrc, pdst, cnts, zeros_tab)
    h2 = _stage_mid(s1, h1, dv, b1r, W2)
    s2 = _sc_layer(h2, psrc, pdst, cnts, zeros_tab)
    h3 = _stage_mid(s2, h2, dv, b2r, W3)
    s3 = _sc_layer(h3, psrc, pdst, cnts, zeros_tab)
    return _stage_final(s3, h3, dv, b3r, Wm1, bm1r, Wm2, bm2r)
